# baseline (speedup 1.0000x reference)
import functools

import jax
import jax.numpy as jnp
from jax import lax
from jax.experimental import pallas as pl
from jax.experimental.pallas import tpu as pltpu
from jax.experimental.pallas import tpu_sc as plsc

N = 4096
M = 200
DIM = 64
NW = 32
IT = 128
L = 16
UNROLL = 1


def _make_sc_gather():
    mesh = plsc.VectorSubcoreMesh(core_axis_name="c", subcore_axis_name="s")

    @functools.partial(
        pl.kernel,
        mesh=mesh,
        out_type=jax.ShapeDtypeStruct((M * (DIM // 8), N // IT, 8, IT), jnp.float32),
        scratch_types=[
            pltpu.VMEM((M, IT), jnp.int32),
            pltpu.VMEM((2, IT, DIM), jnp.float32),
            pltpu.VMEM((2, DIM, IT + 1), jnp.float32),
            pltpu.SemaphoreType.DMA,
            pltpu.SemaphoreType.DMA,
            pltpu.SemaphoreType.DMA,
            pltpu.SemaphoreType.DMA,
        ],
        compiler_params=pltpu.CompilerParams(
            use_tc_tiling_on_sc=False, needs_layout_passes=False
        ),
    )
    def gather_kernel(xt_hbm, table_hbm, out_hbm, xblk_v, g_v, t_v,
                      gsem0, gsem1, wsem0, wsem1):
        wid = lax.axis_index("s") * 2 + lax.axis_index("c")
        gsems = (gsem0, gsem1)
        wsems = (wsem0, wsem1)

        pltpu.sync_copy(xt_hbm.at[:, pl.ds(wid * IT, IT)], xblk_v)

        def gather(j, b):
            return pltpu.make_async_copy(
                table_hbm.at[xblk_v.at[j]], g_v.at[b], gsems[b]
            )

        def write_dt(j, b, dt):
            return pltpu.make_async_copy(
                t_v.at[b, pl.ds(dt * 8, 8), pl.ds(0, IT)],
                out_hbm.at[j * (DIM // 8) + dt, wid],
                wsems[b],
            )

        def write_start(j, b):
            for dt in range(DIM // 8):
                write_dt(j, b, dt).start()

        def write_wait(j, b):
            for dt in range(DIM // 8):
                write_dt(j, b, dt).wait()

        iota = lax.iota(jnp.int32, L)
        d_c = [iota + (L * k) for k in range(DIM // L)]

        def transpose(b):
            @plsc.parallel_loop(0, IT, step=UNROLL, unroll=1)
            def tr_body(i0):
                for u in range(UNROLL):
                    i = i0 + u
                    il = jnp.full((L,), 0, jnp.int32) + i
                    for k in range(DIM // L):
                        v = g_v[b, i, pl.ds(L * k, L)]
                        plsc.store_scatter(t_v.at[b], [d_c[k], il], v)

        gather(0, 0).start()

        def body(jj, carry):
            for b in (0, 1):
                j = 2 * jj + b

                @pl.when(j + 1 < M)
                def _():
                    gather(j + 1, 1 - b).start()

                gather(j, b).wait()

                @pl.when(j >= 2)
                def _():
                    write_wait(j - 2, b)

                transpose(b)
                write_start(j, b)
            return carry

        lax.fori_loop(0, M // 2, body, 0)
        write_wait(M - 2, 0)
        write_wait(M - 1, 1)

    return gather_kernel


_sc_gather = _make_sc_gather()


@jax.jit
def kernel(x, E_relative_position):
    xt = x.astype(jnp.int32).T
    p = _sc_gather(xt, E_relative_position)
    return (
        p.reshape(M, DIM // 8, N // IT, 8, IT)
        .transpose(2, 4, 0, 1, 3)
        .reshape(N, M, DIM)
    )


# speedup vs baseline: 1.0404x; 1.0404x over previous
import functools

import jax
import jax.numpy as jnp
from jax import lax
from jax.experimental import pallas as pl
from jax.experimental.pallas import tpu as pltpu
from jax.experimental.pallas import tpu_sc as plsc

N = 4096
M = 200
DIM = 64
NW = 32
IT = 128
L = 16
UNROLL = 2


def _make_sc_gather():
    mesh = plsc.VectorSubcoreMesh(core_axis_name="c", subcore_axis_name="s")

    @functools.partial(
        pl.kernel,
        mesh=mesh,
        out_type=jax.ShapeDtypeStruct((M, DIM // 8, N // IT, 8, IT), jnp.float32),
        scratch_types=[
            pltpu.VMEM((M, IT), jnp.int32),
            pltpu.VMEM((2, IT, DIM), jnp.float32),
            pltpu.VMEM((2, DIM // 8, 8, IT + 1), jnp.float32),
            pltpu.SemaphoreType.DMA,
            pltpu.SemaphoreType.DMA,
            pltpu.SemaphoreType.DMA,
            pltpu.SemaphoreType.DMA,
        ],
        compiler_params=pltpu.CompilerParams(
            use_tc_tiling_on_sc=False, needs_layout_passes=False
        ),
    )
    def gather_kernel(xt_hbm, table_hbm, out_hbm, xblk_v, g_v, t_v,
                      gsem0, gsem1, wsem0, wsem1):
        wid = lax.axis_index("s") * 2 + lax.axis_index("c")
        gsems = (gsem0, gsem1)
        wsems = (wsem0, wsem1)

        pltpu.sync_copy(xt_hbm.at[:, pl.ds(wid * IT, IT)], xblk_v)

        def gather(j, b):
            return pltpu.make_async_copy(
                table_hbm.at[xblk_v.at[j]], g_v.at[b], gsems[b]
            )

        def write_j(j, b):
            return pltpu.make_async_copy(
                t_v.at[b, :, :, pl.ds(0, IT)],
                out_hbm.at[j, :, wid],
                wsems[b],
            )

        def write_start(j, b):
            write_j(j, b).start()

        def write_wait(j, b):
            write_j(j, b).wait()

        iota = lax.iota(jnp.int32, L)
        dt_c = [(iota + (L * k)) >> 3 for k in range(DIM // L)]
        ds_c = [(iota + (L * k)) & 7 for k in range(DIM // L)]

        def transpose(b):
            @plsc.parallel_loop(0, IT, step=UNROLL, unroll=1)
            def tr_body(i0):
                for u in range(UNROLL):
                    i = i0 + u
                    il = jnp.full((L,), 0, jnp.int32) + i
                    for k in range(DIM // L):
                        v = g_v[b, i, pl.ds(L * k, L)]
                        plsc.store_scatter(t_v.at[b], [dt_c[k], ds_c[k], il], v)

        gather(0, 0).start()

        def body(jj, carry):
            for b in (0, 1):
                j = 2 * jj + b

                @pl.when(j + 1 < M)
                def _():
                    gather(j + 1, 1 - b).start()

                gather(j, b).wait()

                @pl.when(j >= 2)
                def _():
                    write_wait(j - 2, b)

                transpose(b)
                write_start(j, b)
            return carry

        lax.fori_loop(0, M // 2, body, 0)
        write_wait(M - 2, 0)
        write_wait(M - 1, 1)

    return gather_kernel


_sc_gather = _make_sc_gather()


@jax.jit
def kernel(x, E_relative_position):
    xt = x.astype(jnp.int32).T
    p = _sc_gather(xt, E_relative_position)
    return p.transpose(2, 4, 0, 1, 3).reshape(N, M, DIM)


# speedup vs baseline: 1.6980x; 1.6320x over previous
import functools

import jax
import jax.numpy as jnp
from jax import lax
from jax.experimental import pallas as pl
from jax.experimental.pallas import tpu as pltpu
from jax.experimental.pallas import tpu_sc as plsc

N = 4096
M = 200
DIM = 64
NW = 32
IT = 128
L = 16
UNROLL = 2


def _make_sc_gather():
    mesh = plsc.VectorSubcoreMesh(core_axis_name="c", subcore_axis_name="s")

    @functools.partial(
        pl.kernel,
        mesh=mesh,
        out_type=jax.ShapeDtypeStruct((M, DIM // 8, N // IT, 8, IT), jnp.float32),
        scratch_types=[
            pltpu.VMEM((M, IT), jnp.int32),
            pltpu.VMEM((2, IT, DIM), jnp.float32),
            pltpu.VMEM((2, DIM // 8, 8, IT + 1), jnp.float32),
            pltpu.VMEM_SHARED((N, DIM), jnp.float32),
            pltpu.SemaphoreType.DMA,
            pltpu.SemaphoreType.DMA,
            pltpu.SemaphoreType.DMA,
            pltpu.SemaphoreType.DMA,
        ],
        compiler_params=pltpu.CompilerParams(
            use_tc_tiling_on_sc=False, needs_layout_passes=False
        ),
    )
    def gather_kernel(xt_hbm, table_hbm, out_hbm, xblk_v, g_v, t_v,
                      tbl_s, gsem0, gsem1, wsem0, wsem1):
        wid = lax.axis_index("s") * 2 + lax.axis_index("c")
        gsems = (gsem0, gsem1)
        wsems = (wsem0, wsem1)

        @pl.when(lax.axis_index("s") == 0)
        def _():
            pltpu.sync_copy(table_hbm, tbl_s)

        pltpu.sync_copy(xt_hbm.at[:, pl.ds(wid * IT, IT)], xblk_v)
        plsc.subcore_barrier()

        def gather(j, b):
            return pltpu.make_async_copy(
                tbl_s.at[xblk_v.at[j]], g_v.at[b], gsems[b]
            )

        def write_j(j, b):
            return pltpu.make_async_copy(
                t_v.at[b, :, :, pl.ds(0, IT)],
                out_hbm.at[j, :, wid],
                wsems[b],
            )

        def write_start(j, b):
            write_j(j, b).start()

        def write_wait(j, b):
            write_j(j, b).wait()

        iota = lax.iota(jnp.int32, L)
        dt_c = [(iota + (L * k)) >> 3 for k in range(DIM // L)]
        ds_c = [(iota + (L * k)) & 7 for k in range(DIM // L)]

        def transpose(b):
            @plsc.parallel_loop(0, IT, step=UNROLL, unroll=1)
            def tr_body(i0):
                for u in range(UNROLL):
                    i = i0 + u
                    il = jnp.full((L,), 0, jnp.int32) + i
                    for k in range(DIM // L):
                        v = g_v[b, i, pl.ds(L * k, L)]
                        plsc.store_scatter(t_v.at[b], [dt_c[k], ds_c[k], il], v)

        gather(0, 0).start()

        def body(jj, carry):
            for b in (0, 1):
                j = 2 * jj + b

                @pl.when(j + 1 < M)
                def _():
                    gather(j + 1, 1 - b).start()

                gather(j, b).wait()

                @pl.when(j >= 2)
                def _():
                    write_wait(j - 2, b)

                transpose(b)
                write_start(j, b)
            return carry

        lax.fori_loop(0, M // 2, body, 0)
        write_wait(M - 2, 0)
        write_wait(M - 1, 1)

    return gather_kernel


_sc_gather = _make_sc_gather()


@jax.jit
def kernel(x, E_relative_position):
    xt = x.astype(jnp.int32).T
    p = _sc_gather(xt, E_relative_position)
    return p.transpose(2, 4, 0, 1, 3).reshape(N, M, DIM)
